# baseline (speedup 1.0000x reference)
import jax
import jax.numpy as jnp
from jax.experimental import pallas as pl
from jax.experimental.pallas import tpu as pltpu

_TAU = 0.5
_EPS = 1e-10


def _ccl_body(x_ref, out_ref, colsum_ref, acc_ref, cache_ref):
    phase = pl.program_id(0)
    step = pl.program_id(1)
    nsteps = pl.num_programs(1)
    b = cache_ref.shape[0] // nsteps
    n = cache_ref.shape[1]

    dot = lambda a, c: jax.lax.dot_general(
        a, c, (((1,), (0,)), ((), ())),
        precision=jax.lax.Precision.HIGHEST,
        preferred_element_type=jnp.float32)

    @pl.when(phase == 0)
    def _sums():
        @pl.when(step == 0)
        def _init():
            colsum_ref[...] = jnp.zeros_like(colsum_ref)
            acc_ref[...] = jnp.zeros_like(acc_ref)

        s = jnp.exp(x_ref[...] * (1.0 / _TAU))
        ones_b = jnp.ones((8, b), jnp.float32)
        colsum_ref[...] += dot(ones_b, s)[0:1, :]
        cache_ref[pl.ds(step * b, b), :] = s.astype(jnp.bfloat16)

    @pl.when(phase == 1)
    def _loss():
        @pl.when(step == 0)
        def _recip():
            colsum_ref[...] = 1.0 / (colsum_ref[...] + _EPS)

        s = cache_ref[pl.ds(step * b, b), :].astype(jnp.float32)
        ones_n = jnp.ones((n, 128), jnp.float32)
        rsum = dot(s, ones_n)[:, 0:1]
        rinv = 1.0 / (rsum + _EPS)
        cinv = colsum_ref[...]
        term = jnp.log((1.0 - s * rinv) * (1.0 - s * cinv))
        rows = step * b + jax.lax.broadcasted_iota(jnp.int32, (b, n), 0)
        cols = jax.lax.broadcasted_iota(jnp.int32, (b, n), 1)
        term = jnp.where(rows == cols, 0.0, term)
        acc_ref[...] += dot(term, ones_n)[:, 0:1].sum(axis=0, keepdims=True)

        @pl.when(step == nsteps - 1)
        def _finish():
            out_ref[...] = acc_ref[...] * (-1.0 / n)


def kernel(scores):
    n = scores.shape[0]
    block = 512
    nsteps = n // block
    grid = (2, nsteps)
    out = pl.pallas_call(
        _ccl_body,
        grid=grid,
        in_specs=[pl.BlockSpec(
            (block, n),
            lambda p, i: (jnp.where(p == 0, i, nsteps - 1), 0))],
        out_specs=pl.BlockSpec((1, 1), lambda p, i: (0, 0)),
        out_shape=jax.ShapeDtypeStruct((1, 1), jnp.float32),
        scratch_shapes=[
            pltpu.VMEM((1, n), jnp.float32),
            pltpu.VMEM((1, 1), jnp.float32),
            pltpu.VMEM((n, n), jnp.bfloat16),
        ],
        compiler_params=pltpu.CompilerParams(
            dimension_semantics=("arbitrary", "arbitrary"),
        ),
    )(scores)
    return out[0, 0]


# speedup vs baseline: 4.7269x; 4.7269x over previous
import jax
import jax.numpy as jnp
from jax.experimental import pallas as pl
from jax.experimental.pallas import tpu as pltpu

_TAU = 0.5
_EPS = 1e-10


def _ccl_body(x_ref, out_ref, colsum_ref, acc_ref, cache_ref):
    phase = pl.program_id(0)
    step = pl.program_id(1)
    nsteps = pl.num_programs(1)
    b = cache_ref.shape[0] // nsteps
    n = cache_ref.shape[1]

    @pl.when(phase == 0)
    def _sums():
        @pl.when(step == 0)
        def _init():
            colsum_ref[...] = jnp.zeros_like(colsum_ref)
            acc_ref[...] = jnp.zeros_like(acc_ref)

        s = jnp.exp(x_ref[...] * (1.0 / _TAU))
        colsum_ref[...] += s.sum(axis=0, keepdims=True)
        cache_ref[pl.ds(step * b, b), :] = s.astype(jnp.bfloat16)

    @pl.when(phase == 1)
    def _loss():
        @pl.when(step == 0)
        def _recip():
            colsum_ref[...] = 1.0 / (colsum_ref[...] + _EPS)

        s = cache_ref[pl.ds(step * b, b), :].astype(jnp.float32)
        rsum = s.sum(axis=1, keepdims=True)
        rinv = 1.0 / (rsum + _EPS)
        cinv = colsum_ref[...]
        term = jnp.log((1.0 - s * rinv) * (1.0 - s * cinv))
        rows = step * b + jax.lax.broadcasted_iota(jnp.int32, (b, n), 0)
        cols = jax.lax.broadcasted_iota(jnp.int32, (b, n), 1)
        term = jnp.where(rows == cols, 0.0, term)
        acc_ref[...] += term.sum(axis=1, keepdims=True).sum(axis=0, keepdims=True)

        @pl.when(step == nsteps - 1)
        def _finish():
            out_ref[...] = acc_ref[...] * (-1.0 / n)


def kernel(scores):
    n = scores.shape[0]
    block = 512
    nsteps = n // block
    grid = (2, nsteps)
    out = pl.pallas_call(
        _ccl_body,
        grid=grid,
        in_specs=[pl.BlockSpec(
            (block, n),
            lambda p, i: (jnp.where(p == 0, i, nsteps - 1), 0))],
        out_specs=pl.BlockSpec((1, 1), lambda p, i: (0, 0)),
        out_shape=jax.ShapeDtypeStruct((1, 1), jnp.float32),
        scratch_shapes=[
            pltpu.VMEM((1, n), jnp.float32),
            pltpu.VMEM((1, 1), jnp.float32),
            pltpu.VMEM((n, n), jnp.bfloat16),
        ],
        compiler_params=pltpu.CompilerParams(
            dimension_semantics=("arbitrary", "arbitrary"),
        ),
    )(scores)
    return out[0, 0]


# speedup vs baseline: 5.0181x; 1.0616x over previous
import jax
import jax.numpy as jnp
from jax.experimental import pallas as pl
from jax.experimental.pallas import tpu as pltpu

_TAU = 0.5
_EPS = 1e-10


def _ccl_body(x_ref, out_ref, colsum_ref, acc_ref, eye_ref, cache_ref):
    phase = pl.program_id(0)
    step = pl.program_id(1)
    nsteps = pl.num_programs(1)
    b = eye_ref.shape[0]
    n = cache_ref.shape[1]

    @pl.when(phase == 0)
    def _sums():
        @pl.when(step == 0)
        def _init():
            colsum_ref[...] = jnp.zeros_like(colsum_ref)
            acc_ref[...] = jnp.zeros_like(acc_ref)
            r = jax.lax.broadcasted_iota(jnp.int32, (b, b), 0)
            c = jax.lax.broadcasted_iota(jnp.int32, (b, b), 1)
            eye_ref[...] = jnp.where(r == c, 1.0, 0.0)

        s = jnp.exp(x_ref[...] * (1.0 / _TAU))
        colsum_ref[...] += s.sum(axis=0, keepdims=True)
        cache_ref[pl.ds(step * b, b), :] = s.astype(jnp.bfloat16)

    @pl.when(phase == 1)
    def _loss():
        @pl.when(step == 0)
        def _recip():
            colsum_ref[...] = 1.0 / (colsum_ref[...] + _EPS)

        s = cache_ref[pl.ds(step * b, b), :].astype(jnp.float32)
        rsum = s.sum(axis=1, keepdims=True)
        rinv = 1.0 / (rsum + _EPS)
        cinv = colsum_ref[...]
        term = jnp.log((1.0 - s * rinv) * (1.0 - s * cinv))

        dT = (cache_ref[pl.ds(step * b, b), pl.ds(step * b, b)].astype(jnp.float32)
              * eye_ref[...]).sum(axis=0, keepdims=True)
        rinvT = jnp.swapaxes(rinv, 0, 1)
        cinvT = colsum_ref[0:1, pl.ds(step * b, b)]
        dcorr = jnp.log((1.0 - dT * rinvT) * (1.0 - dT * cinvT))

        acc_ref[...] += (term.sum(axis=1, keepdims=True).sum(axis=0, keepdims=True)
                         - dcorr.sum(axis=1, keepdims=True))

        @pl.when(step == nsteps - 1)
        def _finish():
            out_ref[...] = acc_ref[...] * (-1.0 / n)


def kernel(scores):
    n = scores.shape[0]
    block = 512
    nsteps = n // block
    grid = (2, nsteps)
    out = pl.pallas_call(
        _ccl_body,
        grid=grid,
        in_specs=[pl.BlockSpec(
            (block, n),
            lambda p, i: (jnp.where(p == 0, i, nsteps - 1), 0))],
        out_specs=pl.BlockSpec((1, 1), lambda p, i: (0, 0)),
        out_shape=jax.ShapeDtypeStruct((1, 1), jnp.float32),
        scratch_shapes=[
            pltpu.VMEM((1, n), jnp.float32),
            pltpu.VMEM((1, 1), jnp.float32),
            pltpu.VMEM((block, block), jnp.float32),
            pltpu.VMEM((n, n), jnp.bfloat16),
        ],
        compiler_params=pltpu.CompilerParams(
            dimension_semantics=("arbitrary", "arbitrary"),
        ),
    )(scores)
    return out[0, 0]
